# baseline (speedup 1.0000x reference)
import functools

import jax
import jax.numpy as jnp
from jax import lax
from jax.experimental import pallas as pl
from jax.experimental.pallas import tpu as pltpu
from jax.experimental.pallas import tpu_sc as plsc

_NUM_CORES = 2
_NUM_SUBCORES = 16
_NW = _NUM_CORES * _NUM_SUBCORES
_NIMG = 768
_H = 224
_W = 224

_TC_IMGS = 448
_SC_IMGS = _NIMG - _TC_IMGS
_IMG_PER_W = _SC_IMGS // _NW
_TC_BLOCK = 32


def _relu_block(x_ref, o_ref):
    v = x_ref[...]
    o_ref[...] = v * (v > 0)


def _relu_img_inplace(buf):
    def body(r, carry):
        for c in range(_W // 16):
            v = buf[r, pl.ds(c * 16, 16)]
            buf[r, pl.ds(c * 16, 16)] = jnp.where(v > 0, v, 0.0)
        return carry

    lax.fori_loop(0, _H, body, 0)


@functools.partial(
    pl.kernel,
    mesh=plsc.VectorSubcoreMesh(core_axis_name="c", subcore_axis_name="s"),
    out_type=jax.ShapeDtypeStruct((_SC_IMGS, _H, _W), jnp.float32),
    scratch_types=[
        pltpu.VMEM((_H, _W), jnp.float32),
        pltpu.VMEM((_H, _W), jnp.float32),
        pltpu.SemaphoreType.DMA,
        pltpu.SemaphoreType.DMA,
        pltpu.SemaphoreType.DMA,
        pltpu.SemaphoreType.DMA,
    ],
    compiler_params=pltpu.CompilerParams(use_tc_tiling_on_sc=True),
)
def _sc_relu_kernel(x_hbm, o_hbm, b0, b1, si0, si1, so0, so1):
    wid = lax.axis_index("s") * _NUM_CORES + lax.axis_index("c")
    base = wid * _IMG_PER_W
    bufs = (b0, b1)
    isems = (si0, si1)
    osems = (so0, so1)
    in_h = [None, None]
    out_h = [None, None]
    in_h[0] = pltpu.async_copy(x_hbm.at[_TC_IMGS + base], b0, si0)
    for i in range(_IMG_PER_W):
        b = i % 2
        nb = (i + 1) % 2
        if i + 1 < _IMG_PER_W:
            if out_h[nb] is not None:
                out_h[nb].wait()
            in_h[nb] = pltpu.async_copy(
                x_hbm.at[_TC_IMGS + base + i + 1], bufs[nb], isems[nb])
        in_h[b].wait()
        _relu_img_inplace(bufs[b])
        out_h[b] = pltpu.async_copy(bufs[b], o_hbm.at[base + i], osems[b])
    for b in range(2):
        if out_h[b] is not None:
            out_h[b].wait()


def _tc_relu(x3):
    return pl.pallas_call(
        _relu_block,
        out_shape=jax.ShapeDtypeStruct((_TC_IMGS, _H, _W), x3.dtype),
        grid=(_TC_IMGS // _TC_BLOCK,),
        in_specs=[pl.BlockSpec((_TC_BLOCK, _H, _W), lambda i: (i, 0, 0))],
        out_specs=pl.BlockSpec((_TC_BLOCK, _H, _W), lambda i: (i, 0, 0)),
        compiler_params=pltpu.CompilerParams(vmem_limit_bytes=100 * 1024 * 1024),
    )(x3)


def kernel(x, prototype, inter, channel_indices):
    B, C, H, W = x.shape
    x3 = x.reshape(B * C, H, W)
    tc_out = _tc_relu(x3)
    sc_out = _sc_relu_kernel(x3)
    out = jnp.concatenate([tc_out, sc_out], axis=0)
    return out.reshape(B, C, H, W)
